# Initial kernel scaffold; baseline (speedup 1.0000x reference)
import jax
import jax.numpy as jnp
from jax.experimental import pallas as pl


def kernel(x, edge_index, W, b):
    raise NotImplementedError("write your pallas kernel here")


# speedup vs baseline: 27.3674x; 27.3674x over previous
import functools

import jax
import jax.numpy as jnp
from jax import lax
from jax.experimental import pallas as pl
from jax.experimental.pallas import tpu as pltpu
from jax.experimental.pallas import tpu_sc as plsc

_NC = 2
_NS = 16
_NW = _NC * _NS
_K = 80
_W = 25


def _sc_mesh():
    return plsc.VectorSubcoreMesh(core_axis_name="c", subcore_axis_name="s")


def _deg_partials(dst3, n_pad):
    _, NWIN, _, K = dst3.shape
    zc = n_pad // _NS

    @functools.partial(
        pl.kernel,
        mesh=_sc_mesh(),
        out_type=jax.ShapeDtypeStruct((_NC, n_pad), jnp.float32),
        scratch_types=[
            pltpu.VMEM((_W, K), jnp.int32),
            pltpu.VMEM((K,), jnp.float32),
            pltpu.VMEM((zc,), jnp.float32),
            pltpu.VMEM_SHARED((n_pad,), jnp.float32),
            pltpu.SemaphoreType.DMA,
        ],
    )
    def deg_kernel(dst_hbm, out_hbm, dst_v, ones_v, zb_v, deg_sh, sem):
        c = lax.axis_index("c")
        s = lax.axis_index("s")
        wid = s * _NC + c

        def zfill(i, carry):
            zb_v[pl.ds(i * 16, 16)] = jnp.zeros((16,), jnp.float32)
            return carry

        lax.fori_loop(0, zc // 16, zfill, 0)
        for i in range(K // 16):
            ones_v[pl.ds(i * 16, 16)] = jnp.ones((16,), jnp.float32)

        pltpu.sync_copy(zb_v, deg_sh.at[pl.ds(s * zc, zc)])
        plsc.subcore_barrier()

        def window(w, carry):
            pltpu.async_copy(dst_hbm.at[wid, w], dst_v, sem).wait()

            def body(j, carry2):
                pltpu.sync_copy(ones_v, deg_sh.at[dst_v.at[j]], add=True)
                return carry2

            return lax.fori_loop(0, _W, body, carry)

        lax.fori_loop(0, NWIN, window, 0)
        plsc.subcore_barrier()
        pltpu.sync_copy(deg_sh.at[pl.ds(s * zc, zc)],
                        out_hbm.at[c, pl.ds(s * zc, zc)])

    return deg_kernel(dst3)


def _edge_scatter(src3, dst3, h2, n_pad):
    _, NWIN, _, K = src3.shape
    N, D = h2.shape
    rpt = n_pad // _NS
    zr = 64

    @functools.partial(
        pl.kernel,
        mesh=_sc_mesh(),
        out_type=jax.ShapeDtypeStruct((_NC, n_pad, D), jnp.float32),
        scratch_types=[
            pltpu.VMEM((_W, K), jnp.int32),
            pltpu.VMEM((_W, K), jnp.int32),
            pltpu.VMEM((K, D), jnp.float32),
            pltpu.VMEM((zr, D), jnp.float32),
            pltpu.VMEM_SHARED((n_pad, D), jnp.float32),
            pltpu.SemaphoreType.DMA,
        ],
    )
    def edge_kernel(src_hbm, dst_hbm, h2_hbm, out_hbm,
                    src_v, dst_v, rows_v, zrow_v, acc_sh, sem):
        c = lax.axis_index("c")
        s = lax.axis_index("s")
        wid = s * _NC + c

        def zfill(i, carry):
            for j in range(D // 16):
                zrow_v[i, pl.ds(j * 16, 16)] = jnp.zeros((16,), jnp.float32)
            return carry

        lax.fori_loop(0, zr, zfill, 0)

        def zcopy(k, carry):
            pltpu.sync_copy(zrow_v, acc_sh.at[pl.ds(s * rpt + k * zr, zr)])
            return carry

        lax.fori_loop(0, rpt // zr, zcopy, 0)
        plsc.subcore_barrier()

        def window(w, carry):
            pltpu.async_copy(src_hbm.at[wid, w], src_v, sem).wait()
            pltpu.async_copy(dst_hbm.at[wid, w], dst_v, sem).wait()

            def body(j, carry2):
                pltpu.async_copy(h2_hbm.at[src_v.at[j]], rows_v, sem).wait()
                pltpu.sync_copy(rows_v, acc_sh.at[dst_v.at[j]], add=True)
                return carry2

            return lax.fori_loop(0, _W, body, carry)

        lax.fori_loop(0, NWIN, window, 0)
        plsc.subcore_barrier()
        pltpu.sync_copy(acc_sh.at[pl.ds(s * rpt, rpt)],
                        out_hbm.at[c, pl.ds(s * rpt, rpt)])

    return edge_kernel(src3, dst3, h2)


def _tc_h2(x, W, deg2):
    N, _ = x.shape
    Dout = W.shape[1]

    def body(x_ref, w_ref, deg_ref, h2_ref):
        deg = deg_ref[0] + deg_ref[1] + 1.0
        dis = lax.rsqrt(deg)
        h = jnp.dot(x_ref[...], w_ref[...], preferred_element_type=jnp.float32)
        h2_ref[...] = h * dis

    return pl.pallas_call(
        body, out_shape=jax.ShapeDtypeStruct((N, Dout), jnp.float32),
    )(x, W, deg2)


def _tc_finish(acc, h2, deg2, b2):
    N, D = h2.shape

    def body(acc_ref, h2_ref, deg_ref, b_ref, out_ref):
        deg = deg_ref[0] + deg_ref[1] + 1.0
        dis = lax.rsqrt(deg)
        tot = acc_ref[0, :N, :] + acc_ref[1, :N, :] + h2_ref[...]
        out_ref[...] = jnp.maximum(tot * dis + b_ref[...], 0.0)

    return pl.pallas_call(
        body, out_shape=jax.ShapeDtypeStruct((N, D), jnp.float32),
    )(acc, h2, deg2, b2)


def kernel(x, edge_index, W, b):
    N, _ = x.shape
    Dout = W.shape[1]
    E = edge_index.shape[1]
    assert E % (_NW * _K * _W) == 0, "edge count must tile over 32 workers"
    C = E // (_NW * _K)
    n_pad = ((N + 16 * _NS - 1) // (16 * _NS)) * (16 * _NS)

    ei = edge_index.astype(jnp.int32)
    src3 = ei[0].reshape(_NW, C // _W, _W, _K)
    dst3 = ei[1].reshape(_NW, C // _W, _W, _K)

    deg_p = _deg_partials(dst3, n_pad)
    deg2 = deg_p[:, :N].reshape(_NC, N, 1)
    h2 = _tc_h2(x, W, deg2)
    acc = _edge_scatter(src3, dst3, h2, n_pad)
    return _tc_finish(acc, h2, deg2, b.reshape(1, Dout))


# speedup vs baseline: 41.2765x; 1.5082x over previous
import functools

import jax
import jax.numpy as jnp
from jax import lax
from jax.experimental import pallas as pl
from jax.experimental.pallas import tpu as pltpu
from jax.experimental.pallas import tpu_sc as plsc

_NC = 2
_NS = 16
_NW = _NC * _NS
_K = 125
_W = 16


def _sc_mesh():
    return plsc.VectorSubcoreMesh(core_axis_name="c", subcore_axis_name="s")


def _deg_partials(dst4, n_pad):
    _, NWIN, W, K = dst4.shape
    zc = n_pad // _NS
    kp = ((K + 15) // 16) * 16

    @functools.partial(
        pl.kernel,
        mesh=_sc_mesh(),
        out_type=jax.ShapeDtypeStruct((_NC, n_pad), jnp.float32),
        scratch_types=[
            pltpu.VMEM((W, K), jnp.int32),
            pltpu.VMEM((kp,), jnp.float32),
            pltpu.VMEM((zc,), jnp.float32),
            pltpu.VMEM_SHARED((n_pad,), jnp.float32),
            pltpu.SemaphoreType.DMA,
        ],
    )
    def deg_kernel(dst_hbm, out_hbm, dst_v, ones_v, zb_v, deg_sh, sem):
        c = lax.axis_index("c")
        s = lax.axis_index("s")
        wid = s * _NC + c

        def zfill(i, carry):
            zb_v[pl.ds(i * 16, 16)] = jnp.zeros((16,), jnp.float32)
            return carry

        lax.fori_loop(0, zc // 16, zfill, 0)
        for i in range(kp // 16):
            ones_v[pl.ds(i * 16, 16)] = jnp.ones((16,), jnp.float32)

        pltpu.sync_copy(zb_v, deg_sh.at[pl.ds(s * zc, zc)])
        plsc.subcore_barrier()

        def window(w, carry):
            pltpu.async_copy(dst_hbm.at[wid, w], dst_v, sem).wait()

            def body(j, carry2):
                pltpu.sync_copy(ones_v.at[pl.ds(0, K)],
                                deg_sh.at[dst_v.at[j]], add=True)
                return carry2

            return lax.fori_loop(0, W, body, carry)

        lax.fori_loop(0, NWIN, window, 0)
        plsc.subcore_barrier()
        pltpu.sync_copy(deg_sh.at[pl.ds(s * zc, zc)],
                        out_hbm.at[c, pl.ds(s * zc, zc)])

    return deg_kernel(dst4)


def _edge_scatter(src4, dst4, h2, n_pad):
    _, NWIN, W, K = src4.shape
    N, D = h2.shape
    rpt = n_pad // _NS
    zr = 64

    @functools.partial(
        pl.kernel,
        mesh=_sc_mesh(),
        out_type=jax.ShapeDtypeStruct((_NC, n_pad, D), jnp.float32),
        scratch_types=[
            pltpu.VMEM((W, K), jnp.int32),
            pltpu.VMEM((W, K), jnp.int32),
            pltpu.VMEM((K, D), jnp.float32),
            pltpu.VMEM((K, D), jnp.float32),
            pltpu.VMEM((zr, D), jnp.float32),
            pltpu.VMEM_SHARED((n_pad, D), jnp.float32),
            pltpu.SemaphoreType.DMA,
            pltpu.SemaphoreType.DMA,
            pltpu.SemaphoreType.DMA,
        ],
    )
    def edge_kernel(src_hbm, dst_hbm, h2_hbm, out_hbm,
                    src_v, dst_v, rows_a, rows_b, zrow_v, acc_sh,
                    sem_i, sem_a, sem_b):
        c = lax.axis_index("c")
        s = lax.axis_index("s")
        wid = s * _NC + c

        def zfill(i, carry):
            for j in range(D // 16):
                zrow_v[i, pl.ds(j * 16, 16)] = jnp.zeros((16,), jnp.float32)
            return carry

        lax.fori_loop(0, zr, zfill, 0)

        def zcopy(k, carry):
            pltpu.sync_copy(zrow_v, acc_sh.at[pl.ds(s * rpt + k * zr, zr)])
            return carry

        lax.fori_loop(0, rpt // zr, zcopy, 0)
        plsc.subcore_barrier()

        def gather(j, buf, sem):
            pltpu.async_copy(h2_hbm.at[src_v.at[j]], buf, sem)

        def gwait(buf, sem):
            pltpu.make_async_copy(h2_hbm.at[src_v.at[0]], buf, sem).wait()

        def scatter(j, buf):
            pltpu.sync_copy(buf, acc_sh.at[dst_v.at[j]], add=True)

        def window(w, carry):
            pltpu.async_copy(src_hbm.at[wid, w], src_v, sem_i).wait()
            pltpu.async_copy(dst_hbm.at[wid, w], dst_v, sem_i).wait()
            gather(0, rows_a, sem_a)

            def pair(p, carry2):
                gather(2 * p + 1, rows_b, sem_b)
                gwait(rows_a, sem_a)
                scatter(2 * p, rows_a)

                @pl.when(p < W // 2 - 1)
                def _():
                    gather(2 * p + 2, rows_a, sem_a)

                gwait(rows_b, sem_b)
                scatter(2 * p + 1, rows_b)
                return carry2

            return lax.fori_loop(0, W // 2, pair, carry)

        lax.fori_loop(0, NWIN, window, 0)
        plsc.subcore_barrier()
        pltpu.sync_copy(acc_sh.at[pl.ds(s * rpt, rpt)],
                        out_hbm.at[c, pl.ds(s * rpt, rpt)])

    return edge_kernel(src4, dst4, h2)


def _tc_h2(x, W, deg2):
    N, _ = x.shape
    Dout = W.shape[1]

    def body(x_ref, w_ref, deg_ref, h2_ref):
        deg = deg_ref[0] + deg_ref[1] + 1.0
        dis = lax.rsqrt(deg)
        h = jnp.dot(x_ref[...], w_ref[...], preferred_element_type=jnp.float32)
        h2_ref[...] = h * dis

    return pl.pallas_call(
        body, out_shape=jax.ShapeDtypeStruct((N, Dout), jnp.float32),
    )(x, W, deg2)


def _tc_finish(acc, h2, deg2, b2):
    N, D = h2.shape

    def body(acc_ref, h2_ref, deg_ref, b_ref, out_ref):
        deg = deg_ref[0] + deg_ref[1] + 1.0
        dis = lax.rsqrt(deg)
        tot = acc_ref[0, :N, :] + acc_ref[1, :N, :] + h2_ref[...]
        out_ref[...] = jnp.maximum(tot * dis + b_ref[...], 0.0)

    return pl.pallas_call(
        body, out_shape=jax.ShapeDtypeStruct((N, D), jnp.float32),
    )(acc, h2, deg2, b2)


def kernel(x, edge_index, W, b):
    N, _ = x.shape
    Dout = W.shape[1]
    E = edge_index.shape[1]
    assert E % (_NW * _K * _W) == 0, "edge count must tile over 32 workers"
    nwin = E // (_NW * _K * _W)
    n_pad = ((N + 16 * _NS - 1) // (16 * _NS)) * (16 * _NS)

    ei = edge_index.astype(jnp.int32)
    src4 = ei[0].reshape(_NW, nwin, _W, _K)
    dst4 = ei[1].reshape(_NW, nwin, _W, _K)

    deg_p = _deg_partials(dst4, n_pad)
    deg2 = deg_p[:, :N].reshape(_NC, N, 1)
    h2 = _tc_h2(x, W, deg2)
    acc = _edge_scatter(src4, dst4, h2, n_pad)
    return _tc_finish(acc, h2, deg2, b.reshape(1, Dout))


# speedup vs baseline: 42.5367x; 1.0305x over previous
import functools

import jax
import jax.numpy as jnp
from jax import lax
from jax.experimental import pallas as pl
from jax.experimental.pallas import tpu as pltpu
from jax.experimental.pallas import tpu_sc as plsc

_NC = 2
_NS = 16
_NW = _NC * _NS
_K = 125
_W = 16


def _sc_mesh():
    return plsc.VectorSubcoreMesh(core_axis_name="c", subcore_axis_name="s")


def _deg_partials(dst4, n_pad):
    _, NWIN, W, K = dst4.shape
    zc = n_pad // _NS
    kp = ((K + 15) // 16) * 16

    @functools.partial(
        pl.kernel,
        mesh=_sc_mesh(),
        out_type=jax.ShapeDtypeStruct((_NC, n_pad), jnp.float32),
        scratch_types=[
            pltpu.VMEM((W, K), jnp.int32),
            pltpu.VMEM((kp,), jnp.float32),
            pltpu.VMEM((zc,), jnp.float32),
            pltpu.VMEM_SHARED((n_pad,), jnp.float32),
            pltpu.SemaphoreType.DMA,
        ],
    )
    def deg_kernel(dst_hbm, out_hbm, dst_v, ones_v, zb_v, deg_sh, sem):
        c = lax.axis_index("c")
        s = lax.axis_index("s")
        wid = s * _NC + c

        def zfill(i, carry):
            zb_v[pl.ds(i * 16, 16)] = jnp.zeros((16,), jnp.float32)
            return carry

        lax.fori_loop(0, zc // 16, zfill, 0)
        for i in range(kp // 16):
            ones_v[pl.ds(i * 16, 16)] = jnp.ones((16,), jnp.float32)

        pltpu.sync_copy(zb_v, deg_sh.at[pl.ds(s * zc, zc)])
        plsc.subcore_barrier()

        def window(w, carry):
            pltpu.async_copy(dst_hbm.at[wid, w], dst_v, sem).wait()

            def body(j, carry2):
                pltpu.sync_copy(ones_v.at[pl.ds(0, K)],
                                deg_sh.at[dst_v.at[j]], add=True)
                return carry2

            return lax.fori_loop(0, W, body, carry)

        lax.fori_loop(0, NWIN, window, 0)
        plsc.subcore_barrier()
        pltpu.sync_copy(deg_sh.at[pl.ds(s * zc, zc)],
                        out_hbm.at[c, pl.ds(s * zc, zc)])

    return deg_kernel(dst4)


def _edge_scatter(src4, dst4, h2, n_pad):
    _, NWIN, W, K = src4.shape
    N, D = h2.shape
    rpt = n_pad // _NS
    zr = 32

    @functools.partial(
        pl.kernel,
        mesh=_sc_mesh(),
        out_type=jax.ShapeDtypeStruct((_NC, n_pad, D), jnp.float32),
        scratch_types=[
            pltpu.VMEM((2, W, K), jnp.int32),
            pltpu.VMEM((2, W, K), jnp.int32),
            pltpu.VMEM((K, D), jnp.float32),
            pltpu.VMEM((K, D), jnp.float32),
            pltpu.VMEM((zr, D), jnp.float32),
            pltpu.VMEM_SHARED((n_pad, D), jnp.float32),
            pltpu.SemaphoreType.DMA,
            pltpu.SemaphoreType.DMA,
            pltpu.SemaphoreType.DMA,
        ],
    )
    def edge_kernel(src_hbm, dst_hbm, h2_hbm, out_hbm,
                    src_v, dst_v, rows_a, rows_b, zrow_v, acc_sh,
                    sem_i, sem_a, sem_b):
        c = lax.axis_index("c")
        s = lax.axis_index("s")
        wid = s * _NC + c

        def zfill(i, carry):
            for j in range(D // 16):
                zrow_v[i, pl.ds(j * 16, 16)] = jnp.zeros((16,), jnp.float32)
            return carry

        lax.fori_loop(0, zr, zfill, 0)

        def load_idx(w, slot):
            pltpu.async_copy(src_hbm.at[wid, w], src_v.at[slot], sem_i)
            pltpu.async_copy(dst_hbm.at[wid, w], dst_v.at[slot], sem_i)

        def idx_wait():
            pltpu.make_async_copy(src_hbm.at[wid, 0], src_v.at[0], sem_i).wait()
            pltpu.make_async_copy(dst_hbm.at[wid, 0], dst_v.at[0], sem_i).wait()

        def zcopy(k, carry):
            pltpu.sync_copy(zrow_v, acc_sh.at[pl.ds(s * rpt + k * zr, zr)])
            return carry

        load_idx(0, 0)
        lax.fori_loop(0, rpt // zr, zcopy, 0)
        plsc.subcore_barrier()

        def gather(slot, j, buf, sem):
            pltpu.async_copy(h2_hbm.at[src_v.at[slot, j]], buf, sem)

        def gwait(buf, sem):
            pltpu.make_async_copy(h2_hbm.at[src_v.at[0, 0]], buf, sem).wait()

        def scatter(slot, j, buf):
            pltpu.sync_copy(buf, acc_sh.at[dst_v.at[slot, j]], add=True)

        def window(w, carry):
            slot = w % 2
            idx_wait()

            @pl.when(w < NWIN - 1)
            def _():
                load_idx(w + 1, 1 - slot)

            gather(slot, 0, rows_a, sem_a)

            def pair(p, carry2):
                gather(slot, 2 * p + 1, rows_b, sem_b)
                gwait(rows_a, sem_a)
                scatter(slot, 2 * p, rows_a)

                @pl.when(p < W // 2 - 1)
                def _():
                    gather(slot, 2 * p + 2, rows_a, sem_a)

                gwait(rows_b, sem_b)
                scatter(slot, 2 * p + 1, rows_b)
                return carry2

            return lax.fori_loop(0, W // 2, pair, carry)

        lax.fori_loop(0, NWIN, window, 0)
        plsc.subcore_barrier()
        pltpu.sync_copy(acc_sh.at[pl.ds(s * rpt, rpt)],
                        out_hbm.at[c, pl.ds(s * rpt, rpt)])

    return edge_kernel(src4, dst4, h2)


def _tc_h2(x, W, deg2):
    N, _ = x.shape
    Dout = W.shape[1]

    def body(x_ref, w_ref, deg_ref, h2_ref):
        deg = deg_ref[0] + deg_ref[1] + 1.0
        dis = lax.rsqrt(deg)
        h = jnp.dot(x_ref[...], w_ref[...], preferred_element_type=jnp.float32)
        h2_ref[...] = h * dis

    return pl.pallas_call(
        body, out_shape=jax.ShapeDtypeStruct((N, Dout), jnp.float32),
    )(x, W, deg2)


def _tc_finish(acc, h2, deg2, b2):
    N, D = h2.shape

    def body(acc_ref, h2_ref, deg_ref, b_ref, out_ref):
        deg = deg_ref[0] + deg_ref[1] + 1.0
        dis = lax.rsqrt(deg)
        tot = acc_ref[0, :N, :] + acc_ref[1, :N, :] + h2_ref[...]
        out_ref[...] = jnp.maximum(tot * dis + b_ref[...], 0.0)

    return pl.pallas_call(
        body, out_shape=jax.ShapeDtypeStruct((N, D), jnp.float32),
    )(acc, h2, deg2, b2)


def kernel(x, edge_index, W, b):
    N, _ = x.shape
    Dout = W.shape[1]
    E = edge_index.shape[1]
    assert E % (_NW * _K * _W) == 0, "edge count must tile over 32 workers"
    nwin = E // (_NW * _K * _W)
    n_pad = ((N + 16 * _NS - 1) // (16 * _NS)) * (16 * _NS)

    ei = edge_index.astype(jnp.int32)
    src4 = ei[0].reshape(_NW, nwin, _W, _K)
    dst4 = ei[1].reshape(_NW, nwin, _W, _K)

    deg_p = _deg_partials(dst4, n_pad)
    deg2 = deg_p[:, :N].reshape(_NC, N, 1)
    h2 = _tc_h2(x, W, deg2)
    acc = _edge_scatter(src4, dst4, h2, n_pad)
    return _tc_finish(acc, h2, deg2, b.reshape(1, Dout))
